# baseline (speedup 1.0000x reference)
import functools

import jax
import jax.numpy as jnp
from jax import lax
from jax.experimental import pallas as pl
from jax.experimental.pallas import tpu as pltpu
from jax.experimental.pallas import tpu_sc as plsc

_N_NODES = 10000
_N_GRAPHS = 64
_D_NODE = 128
_D_GLOBAL = 8

_N_WORKERS = 16
_N_PAD = 10240
_CHUNK = _N_PAD // _N_WORKERS
_BINS = 80
_BLK = 1000


def _tc_compress(x_ref, xr_ref, wn_ref, u_ref, ur_ref, wu_ref, b_ref,
                 s_ref, ut_ref):
    d = x_ref[...] - xr_ref[...]
    s_ref[...] = jnp.sum(d * wn_ref[...], axis=1, keepdims=True)

    @pl.when(pl.program_id(0) == 0)
    def _():
        ut_ref[...] = jnp.sum((u_ref[...] - ur_ref[...]) * wu_ref[...],
                              axis=1, keepdims=True) + b_ref[...]


def _sc_pool(s_hbm, batch_hbm, ut_hbm, out_hbm,
             s_v, b_v, sums_v, cnts_v, shared_s, shared_c,
             all_s, all_c, ut_v, out_v):
    wid = lax.axis_index("s")
    base = wid * _CHUNK
    pltpu.sync_copy(s_hbm.at[pl.ds(base, _CHUNK)], s_v)
    pltpu.sync_copy(batch_hbm.at[pl.ds(base, _CHUNK)], b_v)

    zeros = jnp.zeros((16,), jnp.float32)
    for j in range(_BINS // 16):
        sums_v[pl.ds(j * 16, 16)] = zeros
        cnts_v[pl.ds(j * 16, 16)] = zeros

    ones = jnp.ones((16,), jnp.float32)

    def step(i, carry):
        idx = b_v[pl.ds(i * 16, 16)]
        vals = s_v[pl.ds(i * 16, 16)]
        plsc.addupdate_scatter(sums_v, [idx], vals)
        plsc.addupdate_scatter(cnts_v, [idx], ones)
        return carry

    lax.fori_loop(0, _CHUNK // 16, step, 0)

    pltpu.sync_copy(sums_v, shared_s.at[pl.ds(wid * _BINS, _BINS)])
    pltpu.sync_copy(cnts_v, shared_c.at[pl.ds(wid * _BINS, _BINS)])
    plsc.subcore_barrier()

    @pl.when(wid == 0)
    def _():
        pltpu.sync_copy(shared_s, all_s)
        pltpu.sync_copy(shared_c, all_c)
        pltpu.sync_copy(ut_hbm, ut_v)
        for j in range(_N_GRAPHS // 16):
            tot = jnp.zeros((16,), jnp.float32)
            cnt = jnp.zeros((16,), jnp.float32)
            for w in range(_N_WORKERS):
                tot = tot + all_s[pl.ds(w * _BINS + j * 16, 16)]
                cnt = cnt + all_c[pl.ds(w * _BINS + j * 16, 16)]
            mean = tot / jnp.maximum(cnt, 1.0)
            out_v[pl.ds(j * 16, 16)] = mean + ut_v[pl.ds(j * 16, 16)]
        pltpu.sync_copy(out_v, out_hbm)


def kernel(x, x_reactant, edge_index, edge_index_reactant, edge_attr,
           edge_attr_reactant, u, u_reactant, batch, W, b):
    del edge_index, edge_index_reactant, edge_attr, edge_attr_reactant
    u = u.reshape(-1, _D_GLOBAL)
    u_reactant = u_reactant.reshape(-1, _D_GLOBAL)
    wn = W[_D_GLOBAL:_D_GLOBAL + _D_NODE].reshape(1, _D_NODE)
    wu = W[:_D_GLOBAL].reshape(1, _D_GLOBAL)
    b2 = b.reshape(1, 1)

    n_blocks = _N_NODES // _BLK
    s, uterm = pl.pallas_call(
        _tc_compress,
        grid=(n_blocks,),
        in_specs=[
            pl.BlockSpec((_BLK, _D_NODE), lambda i: (i, 0)),
            pl.BlockSpec((_BLK, _D_NODE), lambda i: (i, 0)),
            pl.BlockSpec((1, _D_NODE), lambda i: (0, 0)),
            pl.BlockSpec((_N_GRAPHS, _D_GLOBAL), lambda i: (0, 0)),
            pl.BlockSpec((_N_GRAPHS, _D_GLOBAL), lambda i: (0, 0)),
            pl.BlockSpec((1, _D_GLOBAL), lambda i: (0, 0)),
            pl.BlockSpec((1, 1), lambda i: (0, 0)),
        ],
        out_specs=[
            pl.BlockSpec((_BLK, 1), lambda i: (i, 0)),
            pl.BlockSpec((_N_GRAPHS, 1), lambda i: (0, 0)),
        ],
        out_shape=[
            jax.ShapeDtypeStruct((_N_NODES, 1), jnp.float32),
            jax.ShapeDtypeStruct((_N_GRAPHS, 1), jnp.float32),
        ],
    )(x, x_reactant, wn, u, u_reactant, wu, b2)

    s_pad = jnp.pad(s.reshape(-1), (0, _N_PAD - _N_NODES))
    batch_pad = jnp.pad(batch.astype(jnp.int32), (0, _N_PAD - _N_NODES),
                        constant_values=_N_GRAPHS)

    sc_pool = functools.partial(
        pl.kernel,
        out_type=jax.ShapeDtypeStruct((_N_GRAPHS,), jnp.float32),
        mesh=plsc.VectorSubcoreMesh(core_axis_name="c", subcore_axis_name="s",
                                    num_cores=1),
        compiler_params=pltpu.CompilerParams(needs_layout_passes=False),
        scratch_types=[
            pltpu.VMEM((_CHUNK,), jnp.float32),
            pltpu.VMEM((_CHUNK,), jnp.int32),
            pltpu.VMEM((_BINS,), jnp.float32),
            pltpu.VMEM((_BINS,), jnp.float32),
            pltpu.VMEM_SHARED((_N_WORKERS * _BINS,), jnp.float32),
            pltpu.VMEM_SHARED((_N_WORKERS * _BINS,), jnp.float32),
            pltpu.VMEM((_N_WORKERS * _BINS,), jnp.float32),
            pltpu.VMEM((_N_WORKERS * _BINS,), jnp.float32),
            pltpu.VMEM((_N_GRAPHS,), jnp.float32),
            pltpu.VMEM((_N_GRAPHS,), jnp.float32),
        ],
    )(_sc_pool)

    out = sc_pool(s_pad, batch_pad, uterm.reshape(-1))
    return out.reshape(_N_GRAPHS, 1)


# speedup vs baseline: 1.0528x; 1.0528x over previous
import functools

import jax
import jax.numpy as jnp
from jax import lax
from jax.experimental import pallas as pl
from jax.experimental.pallas import tpu as pltpu
from jax.experimental.pallas import tpu_sc as plsc

_N_NODES = 10000
_N_GRAPHS = 64
_D_NODE = 128
_D_GLOBAL = 8

_N_WORKERS = 16
_N_PAD = 10240
_CHUNK = _N_PAD // _N_WORKERS
_TAIL = _N_NODES - 15 * _CHUNK
_BINS = 80
_BLK = 1000


def _tc_compress(x_ref, xr_ref, wn_ref, u_ref, ur_ref, wu_ref, b_ref,
                 s_ref, ut_ref):
    d = x_ref[...] - xr_ref[...]
    s_ref[...] = jnp.sum(d * wn_ref[...], axis=1, keepdims=True)

    @pl.when(pl.program_id(0) == 0)
    def _():
        ut_ref[...] = jnp.sum((u_ref[...] - ur_ref[...]) * wu_ref[...],
                              axis=1, keepdims=True) + b_ref[...]


def _sc_pool(s_hbm, batch_hbm, ut_hbm, out_hbm,
             s_v, b_v, sums_v, cnts_v, shared_s, shared_c,
             all_s, all_c, ut_v, out_v):
    wid = lax.axis_index("s")
    base = wid * _CHUNK
    pltpu.sync_copy(s_hbm.at[pl.ds(base, _CHUNK)], s_v)

    @pl.when(wid < _N_WORKERS - 1)
    def _():
        pltpu.sync_copy(batch_hbm.at[pl.ds(base, _CHUNK)], b_v)

    @pl.when(wid == _N_WORKERS - 1)
    def _():
        pltpu.sync_copy(batch_hbm.at[pl.ds(15 * _CHUNK, _TAIL)],
                        b_v.at[pl.ds(0, _TAIL)])

    zeros = jnp.zeros((16,), jnp.float32)
    for j in range(_BINS // 16):
        sums_v[pl.ds(j * 16, 16)] = zeros
        cnts_v[pl.ds(j * 16, 16)] = zeros

    ones = jnp.ones((16,), jnp.float32)

    def step(i, carry):
        idx = b_v[pl.ds(i * 16, 16)]
        vals = s_v[pl.ds(i * 16, 16)]
        plsc.addupdate_scatter(sums_v, [idx], vals)
        plsc.addupdate_scatter(cnts_v, [idx], ones)
        return carry

    n_windows = jnp.where(wid == _N_WORKERS - 1, _TAIL // 16, _CHUNK // 16)
    lax.fori_loop(0, n_windows, step, 0)

    pltpu.sync_copy(sums_v, shared_s.at[pl.ds(wid * _BINS, _BINS)])
    pltpu.sync_copy(cnts_v, shared_c.at[pl.ds(wid * _BINS, _BINS)])
    plsc.subcore_barrier()

    @pl.when(wid == 0)
    def _():
        pltpu.sync_copy(shared_s, all_s)
        pltpu.sync_copy(shared_c, all_c)
        pltpu.sync_copy(ut_hbm, ut_v)
        for j in range(_N_GRAPHS // 16):
            tot = jnp.zeros((16,), jnp.float32)
            cnt = jnp.zeros((16,), jnp.float32)
            for w in range(_N_WORKERS):
                tot = tot + all_s[pl.ds(w * _BINS + j * 16, 16)]
                cnt = cnt + all_c[pl.ds(w * _BINS + j * 16, 16)]
            mean = tot / jnp.maximum(cnt, 1.0)
            out_v[pl.ds(j * 16, 16)] = mean + ut_v[pl.ds(j * 16, 16)]
        pltpu.sync_copy(out_v, out_hbm)


def kernel(x, x_reactant, edge_index, edge_index_reactant, edge_attr,
           edge_attr_reactant, u, u_reactant, batch, W, b):
    del edge_index, edge_index_reactant, edge_attr, edge_attr_reactant
    u = u.reshape(-1, _D_GLOBAL)
    u_reactant = u_reactant.reshape(-1, _D_GLOBAL)
    wn = W[_D_GLOBAL:_D_GLOBAL + _D_NODE].reshape(1, _D_NODE)
    wu = W[:_D_GLOBAL].reshape(1, _D_GLOBAL)
    b2 = b.reshape(1, 1)

    n_blocks = _N_NODES // _BLK
    s, uterm = pl.pallas_call(
        _tc_compress,
        grid=(n_blocks,),
        in_specs=[
            pl.BlockSpec((_BLK, _D_NODE), lambda i: (i, 0)),
            pl.BlockSpec((_BLK, _D_NODE), lambda i: (i, 0)),
            pl.BlockSpec((1, _D_NODE), lambda i: (0, 0)),
            pl.BlockSpec((_N_GRAPHS, _D_GLOBAL), lambda i: (0, 0)),
            pl.BlockSpec((_N_GRAPHS, _D_GLOBAL), lambda i: (0, 0)),
            pl.BlockSpec((1, _D_GLOBAL), lambda i: (0, 0)),
            pl.BlockSpec((1, 1), lambda i: (0, 0)),
        ],
        out_specs=[
            pl.BlockSpec((_BLK, 1), lambda i: (i, 0)),
            pl.BlockSpec((_N_GRAPHS, 1), lambda i: (0, 0)),
        ],
        out_shape=[
            jax.ShapeDtypeStruct((_N_PAD, 1), jnp.float32),
            jax.ShapeDtypeStruct((_N_GRAPHS, 1), jnp.float32),
        ],
    )(x, x_reactant, wn, u, u_reactant, wu, b2)

    sc_pool = functools.partial(
        pl.kernel,
        out_type=jax.ShapeDtypeStruct((_N_GRAPHS,), jnp.float32),
        mesh=plsc.VectorSubcoreMesh(core_axis_name="c", subcore_axis_name="s",
                                    num_cores=1),
        compiler_params=pltpu.CompilerParams(needs_layout_passes=False),
        scratch_types=[
            pltpu.VMEM((_CHUNK,), jnp.float32),
            pltpu.VMEM((_CHUNK,), jnp.int32),
            pltpu.VMEM((_BINS,), jnp.float32),
            pltpu.VMEM((_BINS,), jnp.float32),
            pltpu.VMEM_SHARED((_N_WORKERS * _BINS,), jnp.float32),
            pltpu.VMEM_SHARED((_N_WORKERS * _BINS,), jnp.float32),
            pltpu.VMEM((_N_WORKERS * _BINS,), jnp.float32),
            pltpu.VMEM((_N_WORKERS * _BINS,), jnp.float32),
            pltpu.VMEM((_N_GRAPHS,), jnp.float32),
            pltpu.VMEM((_N_GRAPHS,), jnp.float32),
        ],
    )(_sc_pool)

    out = sc_pool(s.reshape(-1), batch.astype(jnp.int32), uterm.reshape(-1))
    return out.reshape(_N_GRAPHS, 1)


# speedup vs baseline: 1.3921x; 1.3223x over previous
import functools

import jax
import jax.numpy as jnp
from jax import lax
from jax.experimental import pallas as pl
from jax.experimental.pallas import tpu as pltpu
from jax.experimental.pallas import tpu_sc as plsc

_N_NODES = 10000
_N_GRAPHS = 64
_D_NODE = 128
_D_GLOBAL = 8

_N_WORKERS = 16
_N_PAD = 10240
_CHUNK = _N_PAD // _N_WORKERS
_TAIL = _N_NODES - 15 * _CHUNK
_BINS = 80


def _tc_compress(x_ref, xr_ref, wn_ref, s_ref):
    d = x_ref[...] - xr_ref[...]
    sv = jnp.sum(d * wn_ref[...], axis=1)
    s_ref[0:78, :] = sv[:9984].reshape(78, 128)
    s_ref[78:79, 0:16] = sv[9984:10000].reshape(1, 16)


def _sc_pool(s_hbm, batch_hbm, u_hbm, ur_hbm, w_hbm, b_hbm, out_hbm,
             s_v, b_v, sums_v, cnts_v, u_v, ur_v, w16_v, b1_v, ut_v,
             shared_s, shared_c, shared_u, all_s, all_c, all_u, out_v):
    wid = lax.axis_index("s")
    base = wid * _CHUNK
    pltpu.sync_copy(s_hbm.at[pl.ds(base, _CHUNK)], s_v)

    @pl.when(wid < _N_WORKERS - 1)
    def _():
        pltpu.sync_copy(batch_hbm.at[pl.ds(base, _CHUNK)], b_v)

    @pl.when(wid == _N_WORKERS - 1)
    def _():
        pltpu.sync_copy(batch_hbm.at[pl.ds(15 * _CHUNK, _TAIL)],
                        b_v.at[pl.ds(0, _TAIL)])

    zeros = jnp.zeros((16,), jnp.float32)
    for j in range(_BINS // 16):
        sums_v[pl.ds(j * 16, 16)] = zeros
        cnts_v[pl.ds(j * 16, 16)] = zeros

    ones = jnp.ones((16,), jnp.float32)

    def step(i, carry):
        idx = b_v[pl.ds(i * 16, 16)]
        vals = s_v[pl.ds(i * 16, 16)]
        plsc.addupdate_scatter(sums_v, [idx], vals)
        plsc.addupdate_scatter(cnts_v, [idx], ones)
        return carry

    n_windows = jnp.where(wid == _N_WORKERS - 1, _TAIL // 16, _CHUNK // 16)
    lax.fori_loop(0, n_windows, step, 0)

    @pl.when(wid == _N_WORKERS - 1)
    def _():
        pltpu.sync_copy(u_hbm, u_v)
        pltpu.sync_copy(ur_hbm, ur_v)
        pltpu.sync_copy(w_hbm.at[pl.ds(0, 16)], w16_v)
        pltpu.sync_copy(b_hbm, b1_v.at[pl.ds(0, 1)])
        lanes = lax.iota(jnp.int32, 16)
        w16 = w16_v[pl.ds(0, 16)]
        wupat = w16.at[lanes & 7].get(mode="promise_in_bounds")
        for j in range(_BINS // 16):
            ut_v[pl.ds(j * 16, 16)] = zeros
        for j in range(_N_GRAPHS * _D_GLOBAL // 16):
            du = u_v[pl.ds(j * 16, 16)] - ur_v[pl.ds(j * 16, 16)]
            idx = (lanes >> 3) + 2 * j
            plsc.addupdate_scatter(ut_v, [idx], du * wupat)
        b16 = b1_v[pl.ds(0, 16)]
        bvec = b16.at[jnp.zeros((16,), jnp.int32)].get(mode="promise_in_bounds")
        for j in range(_N_GRAPHS // 16):
            ut_v[pl.ds(j * 16, 16)] = ut_v[pl.ds(j * 16, 16)] + bvec
        pltpu.sync_copy(ut_v, shared_u)

    pltpu.sync_copy(sums_v, shared_s.at[pl.ds(wid * _BINS, _BINS)])
    pltpu.sync_copy(cnts_v, shared_c.at[pl.ds(wid * _BINS, _BINS)])
    plsc.subcore_barrier()

    @pl.when(wid == 0)
    def _():
        pltpu.sync_copy(shared_s, all_s)
        pltpu.sync_copy(shared_c, all_c)
        pltpu.sync_copy(shared_u, all_u)
        for j in range(_N_GRAPHS // 16):
            tot = jnp.zeros((16,), jnp.float32)
            cnt = jnp.zeros((16,), jnp.float32)
            for w in range(_N_WORKERS):
                tot = tot + all_s[pl.ds(w * _BINS + j * 16, 16)]
                cnt = cnt + all_c[pl.ds(w * _BINS + j * 16, 16)]
            mean = tot / jnp.maximum(cnt, 1.0)
            out_v[pl.ds(j * 16, 16)] = mean + all_u[pl.ds(j * 16, 16)]
        pltpu.sync_copy(out_v, out_hbm)


def kernel(x, x_reactant, edge_index, edge_index_reactant, edge_attr,
           edge_attr_reactant, u, u_reactant, batch, W, b):
    del edge_index, edge_index_reactant, edge_attr, edge_attr_reactant
    wn = W[_D_GLOBAL:_D_GLOBAL + _D_NODE].reshape(1, _D_NODE)

    s = pl.pallas_call(
        _tc_compress,
        out_shape=jax.ShapeDtypeStruct((_N_PAD // _D_NODE, _D_NODE),
                                       jnp.float32),
    )(x, x_reactant, wn)

    sc_pool = functools.partial(
        pl.kernel,
        out_type=jax.ShapeDtypeStruct((_N_GRAPHS,), jnp.float32),
        mesh=plsc.VectorSubcoreMesh(core_axis_name="c", subcore_axis_name="s",
                                    num_cores=1),
        compiler_params=pltpu.CompilerParams(needs_layout_passes=False),
        scratch_types=[
            pltpu.VMEM((_CHUNK,), jnp.float32),
            pltpu.VMEM((_CHUNK,), jnp.int32),
            pltpu.VMEM((_BINS,), jnp.float32),
            pltpu.VMEM((_BINS,), jnp.float32),
            pltpu.VMEM((_N_GRAPHS * _D_GLOBAL,), jnp.float32),
            pltpu.VMEM((_N_GRAPHS * _D_GLOBAL,), jnp.float32),
            pltpu.VMEM((16,), jnp.float32),
            pltpu.VMEM((16,), jnp.float32),
            pltpu.VMEM((_BINS,), jnp.float32),
            pltpu.VMEM_SHARED((_N_WORKERS * _BINS,), jnp.float32),
            pltpu.VMEM_SHARED((_N_WORKERS * _BINS,), jnp.float32),
            pltpu.VMEM_SHARED((_BINS,), jnp.float32),
            pltpu.VMEM((_N_WORKERS * _BINS,), jnp.float32),
            pltpu.VMEM((_N_WORKERS * _BINS,), jnp.float32),
            pltpu.VMEM((_BINS,), jnp.float32),
            pltpu.VMEM((_N_GRAPHS,), jnp.float32),
        ],
    )(_sc_pool)

    out = sc_pool(s.reshape(-1), batch.astype(jnp.int32),
                  u.reshape(-1), u_reactant.reshape(-1), W.reshape(-1), b)
    return out.reshape(_N_GRAPHS, 1)
